# baseline (speedup 1.0000x reference)
import functools

import jax
import jax.numpy as jnp
from jax.experimental import pallas as pl
from jax.experimental.pallas import tpu as pltpu
from jax.experimental.pallas import tpu_sc as plsc

_EPS = 1e-12


def _sc_gather(table, idx):
    n = idx.shape[1]
    hid = table.shape[1]
    window = 128
    mesh = plsc.VectorSubcoreMesh(core_axis_name="c", subcore_axis_name="s")

    @functools.partial(
        pl.kernel,
        out_type=jax.ShapeDtypeStruct((n, hid), table.dtype),
        mesh=mesh)
    def k(tab_hbm, idx_hbm, out_hbm):
        def body(i_vmem, o_vmem):
            pltpu.sync_copy(tab_hbm.at[i_vmem.at[0]], o_vmem)

        pltpu.emit_pipeline(
            body,
            grid=(n // window,),
            in_specs=[pl.BlockSpec((1, window), index_map=lambda i: (0, i))],
            out_specs=[pl.BlockSpec((window, hid),
                                    index_map=lambda i: (i, 0))],
            core_axis_name=("c", "s"),
            dimension_semantics=(pltpu.PARALLEL,),
        )(idx_hbm, out_hbm)

    return k(table, idx)


def _ln_body(g_ref, sid_ref, spk_ref, pos_ref, ga_ref, be_ref, o_ref,
             *, bb, seq, hid, nspk):
    x = g_ref[...].astype(jnp.float32) + pos_ref[...][None]

    sid = sid_ref[0, 0, :]
    iota_n = jax.lax.broadcasted_iota(jnp.int32, (bb, nspk), 1)
    m = (sid[:, None] == iota_n).astype(jnp.float32)
    spk_sel = jnp.sum(m[:, :, None] * spk_ref[...][None, :, :], axis=1)
    x = x + spk_sel[:, None, :]

    mu = jnp.mean(x, axis=-1, keepdims=True)
    xc = x - mu
    var = jnp.mean(xc * xc, axis=-1, keepdims=True)
    y = xc * jax.lax.rsqrt(var + _EPS)
    o_ref[...] = y * ga_ref[...][None] + be_ref[...][None]


def _tc_ln(g3, sids, spk_emb, pos_s, gamma, beta, bb):
    batch, seq, hid = g3.shape
    nspk = spk_emb.shape[0]
    nblk = batch // bb
    body = functools.partial(_ln_body, bb=bb, seq=seq, hid=hid, nspk=nspk)
    return pl.pallas_call(
        body,
        grid=(nblk,),
        in_specs=[
            pl.BlockSpec((bb, seq, hid), lambda i: (i, 0, 0)),
            pl.BlockSpec((1, 1, bb), lambda i: (i, 0, 0)),
            pl.BlockSpec((nspk, hid), lambda i: (0, 0)),
            pl.BlockSpec((seq, hid), lambda i: (0, 0)),
            pl.BlockSpec((1, hid), lambda i: (0, 0)),
            pl.BlockSpec((1, hid), lambda i: (0, 0)),
        ],
        out_specs=pl.BlockSpec((bb, seq, hid), lambda i: (i, 0, 0)),
        out_shape=jax.ShapeDtypeStruct((batch, seq, hid), jnp.float32),
    )(g3, sids, spk_emb, pos_s, gamma.reshape(1, hid), beta.reshape(1, hid))


def kernel(input_ids, speaker_ids, char_emb, spk_emb, pos_emb, gamma, beta):
    batch, seq = input_ids.shape
    vocab, hid = char_emb.shape
    bb = 16
    nblk = batch // bb

    idx = input_ids.reshape(1, batch * seq)
    gathered = _sc_gather(char_emb, idx)
    g3 = gathered.reshape(batch, seq, hid)

    pos_s = pos_emb[:seq]
    sids = speaker_ids.reshape(nblk, 1, bb)
    return _tc_ln(g3, sids, spk_emb, pos_s, gamma, beta, bb)


# speedup vs baseline: 1.7190x; 1.7190x over previous
import functools

import jax
import jax.numpy as jnp
from jax.experimental import pallas as pl
from jax.experimental.pallas import tpu as pltpu
from jax.experimental.pallas import tpu_sc as plsc

_EPS = 1e-12
_BB = 16


def _sc_gather(table, idx):
    n = idx.shape[1]
    hid = table.shape[1]
    window = 128
    mesh = plsc.VectorSubcoreMesh(core_axis_name="c", subcore_axis_name="s")

    @functools.partial(
        pl.kernel,
        out_type=jax.ShapeDtypeStruct((n, hid), table.dtype),
        mesh=mesh)
    def k(tab_hbm, idx_hbm, out_hbm):
        def body(i_vmem, o_vmem):
            pltpu.sync_copy(tab_hbm.at[i_vmem.at[0]], o_vmem)

        pltpu.emit_pipeline(
            body,
            grid=(n // window,),
            in_specs=[pl.BlockSpec((1, window), index_map=lambda i: (0, i))],
            out_specs=[pl.BlockSpec((window, hid),
                                    index_map=lambda i: (i, 0))],
            core_axis_name=("c", "s"),
            dimension_semantics=(pltpu.PARALLEL,),
        )(idx_hbm, out_hbm)

    return k(table, idx)


def _spk_pos_ln(x, sid, spk, pos, ga, be, bb, nspk):
    x = x + pos[None]
    iota_n = jax.lax.broadcasted_iota(jnp.int32, (bb, nspk), 1)
    m = (sid[:, None] == iota_n).astype(jnp.float32)
    spk_sel = jnp.sum(m[:, :, None] * spk[None, :, :], axis=1)
    x = x + spk_sel[:, None, :]
    mu = jnp.mean(x, axis=-1, keepdims=True)
    xc = x - mu
    var = jnp.mean(xc * xc, axis=-1, keepdims=True)
    y = xc * jax.lax.rsqrt(var + _EPS)
    return y * ga[None] + be[None]


def _fused_body(ids_ref, sid_ref, char_ref, spk_ref, pos_ref, g_ref, b_ref,
                o_ref, *, bb, seq, vocab, hid, nspk):
    ids = ids_ref[...]
    iota_v = jax.lax.broadcasted_iota(jnp.int32, (bb, seq, vocab), 2)
    oh = (ids[:, :, None] == iota_v).astype(jnp.float32)
    oh = oh.reshape(bb * seq, vocab).astype(jnp.bfloat16)
    emb = jax.lax.dot_general(
        oh, char_ref[...], (((1,), (0,)), ((), ())),
        preferred_element_type=jnp.float32)
    x = emb.reshape(bb, seq, hid)
    o_ref[...] = _spk_pos_ln(x, sid_ref[0, 0, :], spk_ref[...], pos_ref[...],
                             g_ref[...], b_ref[...], bb, nspk)


def _ln_body(prev_ref, g3_ref, sid_ref, spk_ref, pos_ref, ga_ref, be_ref,
             o_ref, *, bb, nspk):
    del prev_ref
    x = g3_ref[...].astype(jnp.float32)
    o_ref[...] = _spk_pos_ln(x, sid_ref[0, 0, :], spk_ref[...], pos_ref[...],
                             ga_ref[...], be_ref[...], bb, nspk)


def kernel(input_ids, speaker_ids, char_emb, spk_emb, pos_emb, gamma, beta):
    batch, seq = input_ids.shape
    vocab, hid = char_emb.shape
    nspk = spk_emb.shape[0]
    bb = _BB
    b_tc = batch // 2
    b_sc = batch - b_tc
    n_tc = b_tc // bb
    n_sc = b_sc // bb

    char_bf = char_emb.astype(jnp.bfloat16)
    pos_s = pos_emb[:seq]
    ga = gamma.reshape(1, hid)
    be = beta.reshape(1, hid)
    sids_tc = speaker_ids[:b_tc].reshape(n_tc, 1, bb)
    sids_sc = speaker_ids[b_tc:].reshape(n_sc, 1, bb)

    fused = functools.partial(_fused_body, bb=bb, seq=seq, vocab=vocab,
                              hid=hid, nspk=nspk)
    out1 = pl.pallas_call(
        fused,
        grid=(n_tc,),
        in_specs=[
            pl.BlockSpec((bb, seq), lambda i: (i, 0)),
            pl.BlockSpec((1, 1, bb), lambda i: (i, 0, 0)),
            pl.BlockSpec((vocab, hid), lambda i: (0, 0)),
            pl.BlockSpec((nspk, hid), lambda i: (0, 0)),
            pl.BlockSpec((seq, hid), lambda i: (0, 0)),
            pl.BlockSpec((1, hid), lambda i: (0, 0)),
            pl.BlockSpec((1, hid), lambda i: (0, 0)),
        ],
        out_specs=pl.BlockSpec((bb, seq, hid), lambda i: (i, 0, 0)),
        out_shape=jax.ShapeDtypeStruct((batch, seq, hid), jnp.float32),
    )(input_ids[:b_tc], sids_tc, char_bf, spk_emb, pos_s, ga, be)

    idx = input_ids[b_tc:].reshape(1, b_sc * seq)
    g3 = _sc_gather(char_emb, idx).reshape(b_sc, seq, hid)

    lnb = functools.partial(_ln_body, bb=bb, nspk=nspk)
    return pl.pallas_call(
        lnb,
        grid=(n_sc,),
        in_specs=[
            pl.BlockSpec((1, 8, hid), lambda i: (0, 0, 0)),
            pl.BlockSpec((bb, seq, hid), lambda i: (i, 0, 0)),
            pl.BlockSpec((1, 1, bb), lambda i: (i, 0, 0)),
            pl.BlockSpec((nspk, hid), lambda i: (0, 0)),
            pl.BlockSpec((seq, hid), lambda i: (0, 0)),
            pl.BlockSpec((1, hid), lambda i: (0, 0)),
            pl.BlockSpec((1, hid), lambda i: (0, 0)),
        ],
        out_specs=pl.BlockSpec((bb, seq, hid),
                               lambda i, n_tc=n_tc: (i + n_tc, 0, 0)),
        out_shape=jax.ShapeDtypeStruct((batch, seq, hid), jnp.float32),
        input_output_aliases={0: 0},
    )(out1, g3, sids_sc, spk_emb, pos_s, ga, be)


# speedup vs baseline: 1.9129x; 1.1128x over previous
import functools

import jax
import jax.numpy as jnp
from jax.experimental import pallas as pl

_EPS = 1e-12


def _body(ids_ref, sid_ref, char_ref, spk_ref, pos_ref, g_ref, b_ref, o_ref,
          *, bb, seq, vocab, hid, nspk):
    ids = ids_ref[...]
    iota_v = jax.lax.broadcasted_iota(jnp.int32, (bb, seq, vocab), 2)
    oh = (ids[:, :, None] == iota_v).astype(jnp.float32)
    oh = oh.reshape(bb * seq, vocab).astype(jnp.bfloat16)
    emb = jax.lax.dot_general(
        oh, char_ref[...], (((1,), (0,)), ((), ())),
        preferred_element_type=jnp.float32)
    x = emb.reshape(bb, seq, hid) + pos_ref[...][None]

    sid = sid_ref[0, 0, :]
    iota_n = jax.lax.broadcasted_iota(jnp.int32, (bb, nspk), 1)
    m = (sid[:, None] == iota_n).astype(jnp.float32)
    spk_sel = jnp.sum(m[:, :, None] * spk_ref[...][None, :, :], axis=1)
    x = x + spk_sel[:, None, :]

    mu = jnp.mean(x, axis=-1, keepdims=True)
    xc = x - mu
    var = jnp.mean(xc * xc, axis=-1, keepdims=True)
    y = xc * jax.lax.rsqrt(var + _EPS)
    o_ref[...] = y * g_ref[...][None] + b_ref[...][None]


def kernel(input_ids, speaker_ids, char_emb, spk_emb, pos_emb, gamma, beta):
    batch, seq = input_ids.shape
    vocab, hid = char_emb.shape
    nspk = spk_emb.shape[0]
    bb = 32
    nblk = batch // bb

    char_bf = char_emb.astype(jnp.bfloat16)
    pos_s = pos_emb[:seq]
    sids = speaker_ids.reshape(nblk, 1, bb)

    body = functools.partial(_body, bb=bb, seq=seq, vocab=vocab, hid=hid,
                             nspk=nspk)

    return pl.pallas_call(
        body,
        grid=(nblk,),
        in_specs=[
            pl.BlockSpec((bb, seq), lambda i: (i, 0)),
            pl.BlockSpec((1, 1, bb), lambda i: (i, 0, 0)),
            pl.BlockSpec((vocab, hid), lambda i: (0, 0)),
            pl.BlockSpec((nspk, hid), lambda i: (0, 0)),
            pl.BlockSpec((seq, hid), lambda i: (0, 0)),
            pl.BlockSpec((1, hid), lambda i: (0, 0)),
            pl.BlockSpec((1, hid), lambda i: (0, 0)),
        ],
        out_specs=pl.BlockSpec((bb, seq, hid), lambda i: (i, 0, 0)),
        out_shape=jax.ShapeDtypeStruct((batch, seq, hid), jnp.float32),
    )(input_ids, sids, char_bf, spk_emb, pos_s,
      gamma.reshape(1, hid), beta.reshape(1, hid))


# speedup vs baseline: 1.9834x; 1.0369x over previous
import functools

import jax
import jax.numpy as jnp
from jax.experimental import pallas as pl

_EPS = 1e-12


def _body(ids_ref, sid_ref, char_ref, spk_ref, pos_ref, g_ref, b_ref, o_ref,
          *, bb, seq, vocab, hid, nspk):
    ids = ids_ref[...]
    iota_v = jax.lax.broadcasted_iota(jnp.int32, (bb, seq, vocab), 2)
    oh = (ids[:, :, None] == iota_v).astype(jnp.float32)
    oh = oh.reshape(bb * seq, vocab).astype(jnp.bfloat16)
    emb = jax.lax.dot_general(
        oh, char_ref[...], (((1,), (0,)), ((), ())),
        preferred_element_type=jnp.float32)
    x = emb.reshape(bb, seq, hid) + pos_ref[...][None]

    sid = sid_ref[0, 0, :]
    iota_n = jax.lax.broadcasted_iota(jnp.int32, (bb, nspk), 1)
    m = (sid[:, None] == iota_n).astype(jnp.float32)
    spk_sel = jnp.sum(m[:, :, None] * spk_ref[...][None, :, :], axis=1)
    x = x + spk_sel[:, None, :]

    mu = jnp.mean(x, axis=-1, keepdims=True)
    xc = x - mu
    var = jnp.mean(xc * xc, axis=-1, keepdims=True)
    y = xc * jax.lax.rsqrt(var + _EPS)
    o_ref[...] = y * g_ref[...][None] + b_ref[...][None]


def kernel(input_ids, speaker_ids, char_emb, spk_emb, pos_emb, gamma, beta):
    batch, seq = input_ids.shape
    vocab, hid = char_emb.shape
    nspk = spk_emb.shape[0]
    bb = 64
    nblk = batch // bb

    char_bf = char_emb.astype(jnp.bfloat16)
    pos_s = pos_emb[:seq]
    sids = speaker_ids.reshape(nblk, 1, bb)

    body = functools.partial(_body, bb=bb, seq=seq, vocab=vocab, hid=hid,
                             nspk=nspk)

    return pl.pallas_call(
        body,
        grid=(nblk,),
        in_specs=[
            pl.BlockSpec((bb, seq), lambda i: (i, 0)),
            pl.BlockSpec((1, 1, bb), lambda i: (i, 0, 0)),
            pl.BlockSpec((vocab, hid), lambda i: (0, 0)),
            pl.BlockSpec((nspk, hid), lambda i: (0, 0)),
            pl.BlockSpec((seq, hid), lambda i: (0, 0)),
            pl.BlockSpec((1, hid), lambda i: (0, 0)),
            pl.BlockSpec((1, hid), lambda i: (0, 0)),
        ],
        out_specs=pl.BlockSpec((bb, seq, hid), lambda i: (i, 0, 0)),
        out_shape=jax.ShapeDtypeStruct((batch, seq, hid), jnp.float32),
    )(input_ids, sids, char_bf, spk_emb, pos_s,
      gamma.reshape(1, hid), beta.reshape(1, hid))


# speedup vs baseline: 2.0244x; 1.0207x over previous
import functools

import jax
import jax.numpy as jnp
from jax.experimental import pallas as pl

_EPS = 1e-12


def _body(ids_ref, sid_ref, char_ref, spk_ref, pos_ref, g_ref, b_ref, o_ref,
          *, bb, seq, vocab, hid, nspk):
    ids = ids_ref[...]
    iota_v = jax.lax.broadcasted_iota(jnp.int32, (bb, seq, vocab), 2)
    oh = (ids[:, :, None] == iota_v).astype(jnp.float32)
    oh = oh.reshape(bb * seq, vocab).astype(jnp.bfloat16)
    emb = jax.lax.dot_general(
        oh, char_ref[...], (((1,), (0,)), ((), ())),
        preferred_element_type=jnp.float32)
    x = emb.reshape(bb, seq, hid) + pos_ref[...][None]

    sid = sid_ref[0, 0, :]
    iota_n = jax.lax.broadcasted_iota(jnp.int32, (bb, nspk), 1)
    m = (sid[:, None] == iota_n).astype(jnp.float32)
    spk_sel = jnp.sum(m[:, :, None] * spk_ref[...][None, :, :], axis=1)
    x = x + spk_sel[:, None, :]

    mu = jnp.mean(x, axis=-1, keepdims=True)
    xc = x - mu
    var = jnp.mean(xc * xc, axis=-1, keepdims=True)
    y = xc * jax.lax.rsqrt(var + _EPS)
    o_ref[...] = y * g_ref[...][None] + b_ref[...][None]


def kernel(input_ids, speaker_ids, char_emb, spk_emb, pos_emb, gamma, beta):
    batch, seq = input_ids.shape
    vocab, hid = char_emb.shape
    nspk = spk_emb.shape[0]
    bb = 128
    nblk = batch // bb

    char_bf = char_emb.astype(jnp.bfloat16)
    pos_s = pos_emb[:seq]
    sids = speaker_ids.reshape(nblk, 1, bb)

    body = functools.partial(_body, bb=bb, seq=seq, vocab=vocab, hid=hid,
                             nspk=nspk)

    return pl.pallas_call(
        body,
        grid=(nblk,),
        in_specs=[
            pl.BlockSpec((bb, seq), lambda i: (i, 0)),
            pl.BlockSpec((1, 1, bb), lambda i: (i, 0, 0)),
            pl.BlockSpec((vocab, hid), lambda i: (0, 0)),
            pl.BlockSpec((nspk, hid), lambda i: (0, 0)),
            pl.BlockSpec((seq, hid), lambda i: (0, 0)),
            pl.BlockSpec((1, hid), lambda i: (0, 0)),
            pl.BlockSpec((1, hid), lambda i: (0, 0)),
        ],
        out_specs=pl.BlockSpec((bb, seq, hid), lambda i: (i, 0, 0)),
        out_shape=jax.ShapeDtypeStruct((batch, seq, hid), jnp.float32),
    )(input_ids, sids, char_bf, spk_emb, pos_s,
      gamma.reshape(1, hid), beta.reshape(1, hid))
